# baseline (speedup 1.0000x reference)
import functools
import jax
import jax.numpy as jnp
from jax import lax
from jax.experimental import pallas as pl
from jax.experimental.pallas import tpu as pltpu
from jax.experimental.pallas import tpu_sc as plsc

N = 10000
E = 320000
R = 8
D = 128
NB = 1000
N_BLOCKS = N // NB

NUM_CORES = 2
NUM_SUBCORES = 16
NUM_TILES = NUM_CORES * NUM_SUBCORES
E_PER_TILE = E // NUM_TILES
CHUNK = 128
N_CHUNKS = E_PER_TILE // CHUNK
TAIL = E_PER_TILE - N_CHUNKS * CHUNK
N_PAD = 10240
ROWS_PER_TILE = N_PAD // NUM_SUBCORES


def _tables_body(feat_ref, w_ref, q_ref, k_ref, h_ref, qn_ref, kn_ref):
    f = feat_ref[...]
    h = jnp.dot(f, w_ref[0], preferred_element_type=jnp.float32)
    h_ref[...] = h
    qn_ref[...] = jnp.dot(h, q_ref[...], preferred_element_type=jnp.float32)
    kn_ref[...] = jnp.dot(h, k_ref[...], preferred_element_type=jnp.float32)


def _tables(feat, W, Q, K):
    return pl.pallas_call(
        _tables_body,
        grid=(N_BLOCKS, R),
        in_specs=[
            pl.BlockSpec((NB, D), lambda n, r: (n, 0)),
            pl.BlockSpec((1, D, D), lambda n, r: (r, 0, 0)),
            pl.BlockSpec((D, 1), lambda n, r: (0, 0)),
            pl.BlockSpec((D, 1), lambda n, r: (0, 0)),
        ],
        out_specs=[
            pl.BlockSpec((NB, D), lambda n, r: (r * N_BLOCKS + n, 0)),
            pl.BlockSpec((NB, 1), lambda n, r: (r * N_BLOCKS + n, 0)),
            pl.BlockSpec((NB, 1), lambda n, r: (r * N_BLOCKS + n, 0)),
        ],
        out_shape=[
            jax.ShapeDtypeStruct((R * N, D), jnp.float32),
            jax.ShapeDtypeStruct((R * N, 1), jnp.float32),
            jax.ShapeDtypeStruct((R * N, 1), jnp.float32),
        ],
    )(feat, W, Q, K)


def _shift_body(qn_ref, kn_ref, out_ref):
    c = jnp.max(qn_ref[...]) + jnp.max(kn_ref[...])
    out_ref[...] = jnp.full((8, 128), c, jnp.float32)


def _shift(qn, kn):
    out = pl.pallas_call(
        _shift_body,
        out_shape=jax.ShapeDtypeStruct((8, 128), jnp.float32),
    )(qn.reshape(R * N // 128, 128), kn.reshape(R * N // 128, 128))
    return out.reshape(-1)[:16]


def _edge_sc(h_table, qn, kn, src, dst, rt, c16, zeros_rows):
    mesh = plsc.VectorSubcoreMesh(core_axis_name="c", subcore_axis_name="s",
                                  num_cores=NUM_CORES,
                                  num_subcores=NUM_SUBCORES)

    @functools.partial(
        pl.kernel,
        out_type=[
            jax.ShapeDtypeStruct((NUM_CORES, N_PAD, D), jnp.float32),
            jax.ShapeDtypeStruct((NUM_CORES, N_PAD // D, D), jnp.float32),
        ],
        mesh=mesh,
        scratch_types=[
            pltpu.VMEM((CHUNK,), jnp.int32),
            pltpu.VMEM((CHUNK,), jnp.int32),
            pltpu.VMEM((CHUNK,), jnp.int32),
            pltpu.VMEM((CHUNK,), jnp.int32),
            pltpu.VMEM((CHUNK,), jnp.int32),
            pltpu.VMEM((CHUNK,), jnp.int32),
            pltpu.VMEM((CHUNK,), jnp.int32),
            pltpu.VMEM((CHUNK,), jnp.float32),
            pltpu.VMEM((CHUNK,), jnp.float32),
            pltpu.VMEM((CHUNK, D), jnp.float32),
            pltpu.VMEM((CHUNK, D), jnp.float32),
            pltpu.VMEM((CHUNK,), jnp.float32),
            pltpu.VMEM((TAIL,), jnp.int32),
            pltpu.VMEM((TAIL,), jnp.int32),
            pltpu.VMEM((16,), jnp.float32),
            pltpu.VMEM_SHARED((N_PAD, D), jnp.float32),
            pltpu.VMEM_SHARED((N_PAD // D, D), jnp.float32),
            pltpu.SemaphoreType.DMA,
            pltpu.SemaphoreType.DMA,
            pltpu.SemaphoreType.DMA,
        ],
    )
    def k(h_hbm, qn_hbm, kn_hbm, src_hbm, dst_hbm, rt_hbm, c_hbm, zero_hbm,
          outf_hbm, outd_hbm,
          src_v, dst_v, rt_v, idx_s, idx_d, sdst, sdrow,
          qbuf, kbuf, hbuf, dbuf, aexp_v, sdst_t, sdrow_t, cvec,
          acc_sh, dacc_sh,
          sem1, sem2, sem3):
        cid = lax.axis_index("c")
        sid = lax.axis_index("s")
        wid = sid * NUM_CORES + cid
        ebase = pl.multiple_of(wid * E_PER_TILE, 8)

        pltpu.sync_copy(c_hbm, cvec)

        rbase = pl.multiple_of(sid * ROWS_PER_TILE, 8)
        pltpu.sync_copy(zero_hbm, acc_sh.at[pl.ds(rbase, ROWS_PER_TILE)])

        @pl.when(sid < N_PAD // D // 8)
        def _zero_den():
            dbase = pl.multiple_of(sid * 8, 8)
            pltpu.sync_copy(zero_hbm.at[pl.ds(0, 8)],
                            dacc_sh.at[pl.ds(dbase, 8)])
        plsc.subcore_barrier()

        lanes = lax.iota(jnp.int32, 16)
        cv = cvec[...]

        def chunk_body(ci, carry):
            base = pl.multiple_of(ebase + ci * CHUNK, 8)
            ce1 = pltpu.async_copy(src_hbm.at[pl.ds(base, CHUNK)], src_v, sem1)
            ce2 = pltpu.async_copy(dst_hbm.at[pl.ds(base, CHUNK)], dst_v, sem2)
            ce3 = pltpu.async_copy(rt_hbm.at[pl.ds(base, CHUNK)], rt_v, sem3)
            ce1.wait()
            ce2.wait()
            ce3.wait()
            for j in range(CHUNK // 16):
                o = 16 * j
                s16 = src_v[pl.ds(o, 16)]
                d16 = dst_v[pl.ds(o, 16)]
                t16 = rt_v[pl.ds(o, 16)]
                idx_s[pl.ds(o, 16)] = t16 * N + s16
                idx_d[pl.ds(o, 16)] = t16 * N + d16
                sdst[pl.ds(o, 16)] = d16
                sdrow[pl.ds(o, 16)] = lax.shift_right_logical(d16, 7)

            cp1 = pltpu.async_copy(qn_hbm.at[idx_d], qbuf, sem1)
            cp2 = pltpu.async_copy(kn_hbm.at[idx_s], kbuf, sem2)
            cp3 = pltpu.async_copy(h_hbm.at[idx_s], hbuf, sem3)
            cp1.wait()
            cp2.wait()
            cp3.wait()

            for j in range(CHUNK // 16):
                o = 16 * j
                a = qbuf[pl.ds(o, 16)] + kbuf[pl.ds(o, 16)]
                a = jnp.where(a > 0, a, 0.2 * a)
                aexp_v[pl.ds(o, 16)] = jnp.exp(a - cv)

            for j in range(CHUNK // 16):
                a16 = aexp_v[pl.ds(16 * j, 16)]
                d16v = sdst[pl.ds(16 * j, 16)]
                dmod = lax.bitwise_and(d16v, 127)
                for l in range(16):
                    r = 16 * j + l
                    s = a16[l]
                    dm = dmod[l]
                    for cb in range(D // 16):
                        o = 16 * cb
                        hbuf[r, pl.ds(o, 16)] = hbuf[r, pl.ds(o, 16)] * s
                        dbuf[r, pl.ds(o, 16)] = jnp.where(
                            lanes + o == dm, s, 0.0)

            pltpu.sync_copy(hbuf, acc_sh.at[sdst], add=True)
            pltpu.sync_copy(dbuf, dacc_sh.at[sdrow], add=True)
            return carry

        lax.fori_loop(0, N_CHUNKS, chunk_body, 0)

        tbase = pl.multiple_of(ebase + N_CHUNKS * CHUNK, 8)
        te1 = pltpu.async_copy(src_hbm.at[pl.ds(tbase, TAIL)],
                               src_v.at[pl.ds(0, TAIL)], sem1)
        te2 = pltpu.async_copy(dst_hbm.at[pl.ds(tbase, TAIL)],
                               dst_v.at[pl.ds(0, TAIL)], sem2)
        te3 = pltpu.async_copy(rt_hbm.at[pl.ds(tbase, TAIL)],
                               rt_v.at[pl.ds(0, TAIL)], sem3)
        te1.wait()
        te2.wait()
        te3.wait()
        s16 = src_v[pl.ds(0, 16)]
        d16 = dst_v[pl.ds(0, 16)]
        t16 = rt_v[pl.ds(0, 16)]
        idx_s[pl.ds(0, 16)] = t16 * N + s16
        idx_d[pl.ds(0, 16)] = t16 * N + d16
        sdst_t[pl.ds(0, 16)] = d16
        sdrow_t[pl.ds(0, 16)] = lax.shift_right_logical(d16, 7)

        tp1 = pltpu.async_copy(qn_hbm.at[idx_d.at[pl.ds(0, TAIL)]],
                               qbuf.at[pl.ds(0, TAIL)], sem1)
        tp2 = pltpu.async_copy(kn_hbm.at[idx_s.at[pl.ds(0, TAIL)]],
                               kbuf.at[pl.ds(0, TAIL)], sem2)
        tp3 = pltpu.async_copy(h_hbm.at[idx_s.at[pl.ds(0, TAIL)]],
                               hbuf.at[pl.ds(0, TAIL)], sem3)
        tp1.wait()
        tp2.wait()
        tp3.wait()

        a = qbuf[pl.ds(0, 16)] + kbuf[pl.ds(0, 16)]
        a = jnp.where(a > 0, a, 0.2 * a)
        ae16 = jnp.exp(a - cv)
        dmod = lax.bitwise_and(d16, 127)
        for l in range(16):
            s = ae16[l]
            dm = dmod[l]
            for cb in range(D // 16):
                o = 16 * cb
                hbuf[l, pl.ds(o, 16)] = hbuf[l, pl.ds(o, 16)] * s
                dbuf[l, pl.ds(o, 16)] = jnp.where(lanes + o == dm, s, 0.0)

        pltpu.sync_copy(hbuf.at[pl.ds(0, TAIL)], acc_sh.at[sdst_t], add=True)
        pltpu.sync_copy(dbuf.at[pl.ds(0, TAIL)], dacc_sh.at[sdrow_t], add=True)

        plsc.subcore_barrier()
        pltpu.sync_copy(acc_sh.at[pl.ds(rbase, ROWS_PER_TILE)],
                        outf_hbm.at[cid, pl.ds(rbase, ROWS_PER_TILE)])

        @pl.when(sid < N_PAD // D // 8)
        def _write_den():
            dbase = pl.multiple_of(sid * 8, 8)
            pltpu.sync_copy(dacc_sh.at[pl.ds(dbase, 8)],
                            outd_hbm.at[cid, pl.ds(dbase, 8)])

    return k(h_table, qn, kn, src, dst, rt, c16, zeros_rows)


def _finalize_body(acc_ref, den_ref, b_ref, out_ref):
    acc = acc_ref[...]
    den = den_ref[...]
    top = acc[0] + acc[1]
    d = den[0] + den[1]
    out_ref[...] = jnp.maximum(top / (d + 1e-16) + b_ref[...], 0.0)


def _finalize(accf, accd, b):
    return pl.pallas_call(
        _finalize_body,
        grid=(N_BLOCKS,),
        in_specs=[
            pl.BlockSpec((NUM_CORES, NB, D), lambda n: (0, n, 0)),
            pl.BlockSpec((NUM_CORES, NB, 1), lambda n: (0, n, 0)),
            pl.BlockSpec((1, D), lambda n: (0, 0)),
        ],
        out_specs=pl.BlockSpec((NB, D), lambda n: (n, 0)),
        out_shape=jax.ShapeDtypeStruct((N, D), jnp.float32),
    )(accf, accd, b.reshape(1, D))


def _head_body(feat_ref, w_ref, b_ref, out_ref):
    o = jnp.dot(feat_ref[...], w_ref[...], preferred_element_type=jnp.float32)
    o = o + b_ref[...]
    m = jnp.max(o, axis=-1, keepdims=True)
    e = jnp.exp(o - m)
    out_ref[...] = o - m - jnp.log(jnp.sum(e, axis=-1, keepdims=True))


def _head(feat, lin_W, lin_b):
    d_out = lin_W.shape[1]
    return pl.pallas_call(
        _head_body,
        grid=(N_BLOCKS,),
        in_specs=[
            pl.BlockSpec((NB, D), lambda n: (n, 0)),
            pl.BlockSpec((D, d_out), lambda n: (0, 0)),
            pl.BlockSpec((1, d_out), lambda n: (0, 0)),
        ],
        out_specs=pl.BlockSpec((NB, d_out), lambda n: (n, 0)),
        out_shape=jax.ShapeDtypeStruct((N, d_out), jnp.float32),
    )(feat, lin_W, lin_b.reshape(1, d_out))


def _layer(feat, src, dst, rt, W, Q, K, b, zeros_rows):
    h_table, qn, kn = _tables(feat, W, Q, K)
    qn = qn.reshape(-1)
    kn = kn.reshape(-1)
    c16 = _shift(qn, kn)
    accf, accd = _edge_sc(h_table, qn, kn, src, dst, rt, c16, zeros_rows)
    den = accd.reshape(NUM_CORES, N_PAD)[:, :N]
    return _finalize(accf[:, :N], den.reshape(NUM_CORES, N, 1), b)


@jax.jit
def kernel(x, edge_index, edge_type, W1, Q1, K1, b1, W2, Q2, K2, b2,
           lin_W, lin_b):
    src = edge_index[0]
    dst = edge_index[1]
    rt = edge_type
    zeros_rows = jnp.zeros((ROWS_PER_TILE, D), jnp.float32)
    h = _layer(x, src, dst, rt, W1, Q1, K1, b1, zeros_rows)
    h = _layer(h, src, dst, rt, W2, Q2, K2, b2, zeros_rows)
    return _head(h, lin_W, lin_b)


# speedup vs baseline: 1.1570x; 1.1570x over previous
import functools
import jax
import jax.numpy as jnp
from jax import lax
from jax.experimental import pallas as pl
from jax.experimental.pallas import tpu as pltpu
from jax.experimental.pallas import tpu_sc as plsc

N = 10000
E = 320000
R = 8
D = 128
NB = 1000
N_BLOCKS = N // NB

NUM_CORES = 2
NUM_SUBCORES = 16
NUM_TILES = NUM_CORES * NUM_SUBCORES
E_PER_TILE = E // NUM_TILES
CHUNK = 80
N_CHUNKS = E_PER_TILE // CHUNK
N_PAD = 10240
ROWS_PER_TILE = N_PAD // NUM_SUBCORES


def _tables_body(feat_ref, w_ref, q_ref, k_ref, h_ref, qn_ref, kn_ref):
    f = feat_ref[...]
    h = jnp.dot(f, w_ref[0], preferred_element_type=jnp.float32)
    h_ref[...] = h
    qn_ref[...] = jnp.dot(h, q_ref[...], preferred_element_type=jnp.float32)
    kn_ref[...] = jnp.dot(h, k_ref[...], preferred_element_type=jnp.float32)


def _tables(feat, W, Q, K):
    return pl.pallas_call(
        _tables_body,
        grid=(N_BLOCKS, R),
        in_specs=[
            pl.BlockSpec((NB, D), lambda n, r: (n, 0)),
            pl.BlockSpec((1, D, D), lambda n, r: (r, 0, 0)),
            pl.BlockSpec((D, 1), lambda n, r: (0, 0)),
            pl.BlockSpec((D, 1), lambda n, r: (0, 0)),
        ],
        out_specs=[
            pl.BlockSpec((NB, D), lambda n, r: (r * N_BLOCKS + n, 0)),
            pl.BlockSpec((NB, 1), lambda n, r: (r * N_BLOCKS + n, 0)),
            pl.BlockSpec((NB, 1), lambda n, r: (r * N_BLOCKS + n, 0)),
        ],
        out_shape=[
            jax.ShapeDtypeStruct((R * N, D), jnp.float32),
            jax.ShapeDtypeStruct((R * N, 1), jnp.float32),
            jax.ShapeDtypeStruct((R * N, 1), jnp.float32),
        ],
    )(feat, W, Q, K)


def _shift_body(qn_ref, kn_ref, out_ref):
    c = jnp.max(qn_ref[...]) + jnp.max(kn_ref[...])
    out_ref[...] = jnp.full((8, 128), c, jnp.float32)


def _shift(qn, kn):
    out = pl.pallas_call(
        _shift_body,
        out_shape=jax.ShapeDtypeStruct((8, 128), jnp.float32),
    )(qn.reshape(R * N // 128, 128), kn.reshape(R * N // 128, 128))
    return out.reshape(-1)[:16]


def _edge_sc(h_table, qn, kn, src, dst, rt, c16, zeros_rows):
    mesh = plsc.VectorSubcoreMesh(core_axis_name="c", subcore_axis_name="s",
                                  num_cores=NUM_CORES,
                                  num_subcores=NUM_SUBCORES)

    @functools.partial(
        pl.kernel,
        out_type=[
            jax.ShapeDtypeStruct((NUM_CORES, N_PAD, D), jnp.float32),
            jax.ShapeDtypeStruct((NUM_CORES, N_PAD // D, D), jnp.float32),
        ],
        mesh=mesh,
        scratch_types=[
            pltpu.VMEM((CHUNK,), jnp.int32),
            pltpu.VMEM((CHUNK,), jnp.int32),
            pltpu.VMEM((CHUNK,), jnp.int32),
            pltpu.VMEM((CHUNK,), jnp.int32),
            pltpu.VMEM((CHUNK,), jnp.int32),
            pltpu.VMEM((CHUNK,), jnp.int32),
            pltpu.VMEM((CHUNK,), jnp.int32),
            pltpu.VMEM((CHUNK,), jnp.float32),
            pltpu.VMEM((CHUNK,), jnp.float32),
            pltpu.VMEM((CHUNK, D), jnp.float32),
            pltpu.VMEM((CHUNK, D), jnp.float32),
            pltpu.VMEM((CHUNK,), jnp.float32),
            pltpu.VMEM((16,), jnp.float32),
            pltpu.VMEM_SHARED((N_PAD, D), jnp.float32),
            pltpu.VMEM_SHARED((N_PAD // D, D), jnp.float32),
            pltpu.SemaphoreType.DMA,
            pltpu.SemaphoreType.DMA,
            pltpu.SemaphoreType.DMA,
        ],
    )
    def k(h_hbm, qn_hbm, kn_hbm, src_hbm, dst_hbm, rt_hbm, c_hbm, zero_hbm,
          outf_hbm, outd_hbm,
          src_v, dst_v, rt_v, idx_s, idx_d, sdst, sdrow,
          qbuf, kbuf, hbuf, dbuf, aexp_v, cvec,
          acc_sh, dacc_sh,
          sem1, sem2, sem3):
        cid = lax.axis_index("c")
        sid = lax.axis_index("s")
        wid = sid * NUM_CORES + cid
        ebase = pl.multiple_of(wid * E_PER_TILE, 8)

        pltpu.sync_copy(c_hbm, cvec)

        rbase = pl.multiple_of(sid * ROWS_PER_TILE, 8)
        pltpu.sync_copy(zero_hbm, acc_sh.at[pl.ds(rbase, ROWS_PER_TILE)])

        @pl.when(sid < N_PAD // D // 8)
        def _zero_den():
            dbase = pl.multiple_of(sid * 8, 8)
            pltpu.sync_copy(zero_hbm.at[pl.ds(0, 8)],
                            dacc_sh.at[pl.ds(dbase, 8)])
        plsc.subcore_barrier()

        lanes = lax.iota(jnp.int32, 16)
        cv = cvec[...]

        def chunk_body(ci, carry):
            base = pl.multiple_of(ebase + ci * CHUNK, 8)
            ce1 = pltpu.async_copy(src_hbm.at[pl.ds(base, CHUNK)], src_v, sem1)
            ce2 = pltpu.async_copy(dst_hbm.at[pl.ds(base, CHUNK)], dst_v, sem2)
            ce3 = pltpu.async_copy(rt_hbm.at[pl.ds(base, CHUNK)], rt_v, sem3)
            ce1.wait()
            ce2.wait()
            ce3.wait()
            for j in range(CHUNK // 16):
                o = 16 * j
                s16 = src_v[pl.ds(o, 16)]
                d16 = dst_v[pl.ds(o, 16)]
                t16 = rt_v[pl.ds(o, 16)]
                idx_s[pl.ds(o, 16)] = t16 * N + s16
                idx_d[pl.ds(o, 16)] = t16 * N + d16
                sdst[pl.ds(o, 16)] = d16
                sdrow[pl.ds(o, 16)] = lax.shift_right_logical(d16, 7)

            cp1 = pltpu.async_copy(qn_hbm.at[idx_d], qbuf, sem1)
            cp2 = pltpu.async_copy(kn_hbm.at[idx_s], kbuf, sem2)
            cp3 = pltpu.async_copy(h_hbm.at[idx_s], hbuf, sem3)
            cp1.wait()
            cp2.wait()
            cp3.wait()

            for j in range(CHUNK // 16):
                o = 16 * j
                a = qbuf[pl.ds(o, 16)] + kbuf[pl.ds(o, 16)]
                a = jnp.where(a > 0, a, 0.2 * a)
                aexp_v[pl.ds(o, 16)] = jnp.exp(a - cv)

            for j in range(CHUNK // 16):
                a16 = aexp_v[pl.ds(16 * j, 16)]
                d16v = sdst[pl.ds(16 * j, 16)]
                dmod = lax.bitwise_and(d16v, 127)
                for l in range(16):
                    r = 16 * j + l
                    s = a16[l]
                    dm = dmod[l]
                    for cb in range(D // 16):
                        o = 16 * cb
                        hbuf[r, pl.ds(o, 16)] = hbuf[r, pl.ds(o, 16)] * s
                        dbuf[r, pl.ds(o, 16)] = jnp.where(
                            lanes + o == dm, s, 0.0)

            pltpu.sync_copy(hbuf, acc_sh.at[sdst], add=True)
            pltpu.sync_copy(dbuf, dacc_sh.at[sdrow], add=True)
            return carry

        lax.fori_loop(0, N_CHUNKS, chunk_body, 0)

        plsc.subcore_barrier()
        pltpu.sync_copy(acc_sh.at[pl.ds(rbase, ROWS_PER_TILE)],
                        outf_hbm.at[cid, pl.ds(rbase, ROWS_PER_TILE)])

        @pl.when(sid < N_PAD // D // 8)
        def _write_den():
            dbase = pl.multiple_of(sid * 8, 8)
            pltpu.sync_copy(dacc_sh.at[pl.ds(dbase, 8)],
                            outd_hbm.at[cid, pl.ds(dbase, 8)])

    return k(h_table, qn, kn, src, dst, rt, c16, zeros_rows)


def _finalize_body(acc_ref, den_ref, b_ref, out_ref):
    acc = acc_ref[...]
    den = den_ref[...]
    top = acc[0] + acc[1]
    d = den[0] + den[1]
    out_ref[...] = jnp.maximum(top / (d + 1e-16) + b_ref[...], 0.0)


def _finalize(accf, accd, b):
    return pl.pallas_call(
        _finalize_body,
        grid=(N_BLOCKS,),
        in_specs=[
            pl.BlockSpec((NUM_CORES, NB, D), lambda n: (0, n, 0)),
            pl.BlockSpec((NUM_CORES, NB, 1), lambda n: (0, n, 0)),
            pl.BlockSpec((1, D), lambda n: (0, 0)),
        ],
        out_specs=pl.BlockSpec((NB, D), lambda n: (n, 0)),
        out_shape=jax.ShapeDtypeStruct((N, D), jnp.float32),
    )(accf, accd, b.reshape(1, D))


def _head_body(feat_ref, w_ref, b_ref, out_ref):
    o = jnp.dot(feat_ref[...], w_ref[...], preferred_element_type=jnp.float32)
    o = o + b_ref[...]
    m = jnp.max(o, axis=-1, keepdims=True)
    e = jnp.exp(o - m)
    out_ref[...] = o - m - jnp.log(jnp.sum(e, axis=-1, keepdims=True))


def _head(feat, lin_W, lin_b):
    d_out = lin_W.shape[1]
    return pl.pallas_call(
        _head_body,
        grid=(N_BLOCKS,),
        in_specs=[
            pl.BlockSpec((NB, D), lambda n: (n, 0)),
            pl.BlockSpec((D, d_out), lambda n: (0, 0)),
            pl.BlockSpec((1, d_out), lambda n: (0, 0)),
        ],
        out_specs=pl.BlockSpec((NB, d_out), lambda n: (n, 0)),
        out_shape=jax.ShapeDtypeStruct((N, d_out), jnp.float32),
    )(feat, lin_W, lin_b.reshape(1, d_out))


def _layer(feat, src, dst, rt, W, Q, K, b, zeros_rows):
    h_table, qn, kn = _tables(feat, W, Q, K)
    qn = qn.reshape(-1)
    kn = kn.reshape(-1)
    c16 = _shift(qn, kn)
    accf, accd = _edge_sc(h_table, qn, kn, src, dst, rt, c16, zeros_rows)
    den = accd.reshape(NUM_CORES, N_PAD)[:, :N]
    return _finalize(accf[:, :N], den.reshape(NUM_CORES, N, 1), b)


@jax.jit
def kernel(x, edge_index, edge_type, W1, Q1, K1, b1, W2, Q2, K2, b2,
           lin_W, lin_b):
    src = edge_index[0]
    dst = edge_index[1]
    rt = edge_type
    zeros_rows = jnp.zeros((ROWS_PER_TILE, D), jnp.float32)
    h = _layer(x, src, dst, rt, W1, Q1, K1, b1, zeros_rows)
    h = _layer(h, src, dst, rt, W2, Q2, K2, b2, zeros_rows)
    return _head(h, lin_W, lin_b)
